# baseline (speedup 1.0000x reference)
import functools

import jax
import jax.numpy as jnp
from jax import lax
from jax.experimental import pallas as pl
from jax.experimental.pallas import tpu as pltpu
from jax.experimental.pallas import tpu_sc as plsc

OUT_ROWS = 4096
DIM = 4096
NUM_CORES = 2
NUM_SUBCORES = 16
NUM_WORKERS = NUM_CORES * NUM_SUBCORES
ROWS_PER_W = OUT_ROWS // NUM_WORKERS
CHUNK = 8
NCHUNK = ROWS_PER_W // CHUNK
NBUF = 3


def _sc_body(nst_hbm, table_hbm, out_hbm, idx_v, shared,
             gsem0, gsem1, gsem2, ssem0, ssem1, ssem2):
    sid = lax.axis_index("s")
    wid = sid * NUM_CORES + lax.axis_index("c")
    pltpu.sync_copy(nst_hbm, idx_v.at[pl.ds(0, 1)])
    start = idx_v[...][0] - OUT_ROWS
    base = wid * ROWS_PER_W

    gsems = (gsem0, gsem1, gsem2)
    ssems = (ssem0, ssem1, ssem2)

    def gather(c):
        slot = c % NBUF
        src = pl.multiple_of(start + (base + c * CHUNK), 8)
        return pltpu.make_async_copy(
            table_hbm.at[pl.ds(src, CHUNK)], shared.at[sid, slot], gsems[slot])

    def scatter(c):
        slot = c % NBUF
        return pltpu.make_async_copy(
            shared.at[sid, slot],
            out_hbm.at[pl.ds(base + c * CHUNK, CHUNK)], ssems[slot])

    for b in range(NBUF):
        gather(b).start()
    for c in range(NCHUNK):
        gather(c).wait()
        scatter(c).start()
        if c >= NBUF - 1 and c + 1 < NCHUNK:
            scatter(c - (NBUF - 1)).wait()
            gather(c + 1).start()
    for c in range(NCHUNK - (NBUF - 1), NCHUNK):
        scatter(c).wait()


_sc_copy = functools.partial(
    pl.kernel,
    mesh=plsc.VectorSubcoreMesh(core_axis_name="c", subcore_axis_name="s"),
    out_type=jax.ShapeDtypeStruct((OUT_ROWS, DIM), jnp.float32),
    scratch_types=[
        pltpu.VMEM((16,), jnp.int32),
        pltpu.VMEM_SHARED((NUM_SUBCORES, NBUF, CHUNK, DIM), jnp.float32),
        pltpu.SemaphoreType.DMA,
        pltpu.SemaphoreType.DMA,
        pltpu.SemaphoreType.DMA,
        pltpu.SemaphoreType.DMA,
        pltpu.SemaphoreType.DMA,
        pltpu.SemaphoreType.DMA,
    ],
)(_sc_body)


def kernel(pos_emb_weight, n_timesteps):
    nst = jnp.asarray(n_timesteps, jnp.int32).reshape((1,))
    return _sc_copy(nst, pos_emb_weight)


# speedup vs baseline: 1.0212x; 1.0212x over previous
import functools

import jax
import jax.numpy as jnp
from jax import lax
from jax.experimental import pallas as pl
from jax.experimental.pallas import tpu as pltpu
from jax.experimental.pallas import tpu_sc as plsc

OUT_ROWS = 4096
DIM = 4096
NUM_CORES = 2
NUM_SUBCORES = 16
NUM_WORKERS = NUM_CORES * NUM_SUBCORES
ROWS_PER_W = OUT_ROWS // NUM_WORKERS
CHUNK = 8
NCHUNK = ROWS_PER_W // CHUNK


def _sc_body(nst_hbm, table_hbm, out_hbm, idx_v, shared,
             gsem0, gsem1, ssem0, ssem1):
    sid = lax.axis_index("s")
    wid = sid * NUM_CORES + lax.axis_index("c")
    pltpu.sync_copy(nst_hbm, idx_v.at[pl.ds(0, 1)])
    start = idx_v[...][0] - OUT_ROWS
    base = wid * ROWS_PER_W

    gsems = (gsem0, gsem1)
    ssems = (ssem0, ssem1)

    def gather(c, slot):
        src = pl.multiple_of(start + (base + c * CHUNK), 8)
        return pltpu.make_async_copy(
            table_hbm.at[pl.ds(src, CHUNK)], shared.at[sid, slot], gsems[slot])

    def scatter(c, slot):
        return pltpu.make_async_copy(
            shared.at[sid, slot],
            out_hbm.at[pl.ds(base + c * CHUNK, CHUNK)], ssems[slot])

    gather(0, 0).start()
    gather(1, 1).start()
    gather(0, 0).wait()
    scatter(0, 0).start()

    def body(g, carry):
        c1 = 2 * g + 1
        gather(c1, 1).wait()
        scatter(c1, 1).start()
        scatter(c1 - 1, 0).wait()
        gather(c1 + 1, 0).start()
        c2 = 2 * g + 2
        gather(c2, 0).wait()
        scatter(c2, 0).start()
        scatter(c2 - 1, 1).wait()
        gather(c2 + 1, 1).start()
        return carry

    lax.fori_loop(0, (NCHUNK - 2) // 2, body, 0)

    gather(NCHUNK - 1, 1).wait()
    scatter(NCHUNK - 1, 1).start()
    scatter(NCHUNK - 2, 0).wait()
    scatter(NCHUNK - 1, 1).wait()


_sc_copy = functools.partial(
    pl.kernel,
    mesh=plsc.VectorSubcoreMesh(core_axis_name="c", subcore_axis_name="s"),
    out_type=jax.ShapeDtypeStruct((OUT_ROWS, DIM), jnp.float32),
    scratch_types=[
        pltpu.VMEM((16,), jnp.int32),
        pltpu.VMEM_SHARED((NUM_SUBCORES, 2, CHUNK, DIM), jnp.float32),
        pltpu.SemaphoreType.DMA,
        pltpu.SemaphoreType.DMA,
        pltpu.SemaphoreType.DMA,
        pltpu.SemaphoreType.DMA,
    ],
)(_sc_body)


def kernel(pos_emb_weight, n_timesteps):
    nst = jnp.asarray(n_timesteps, jnp.int32).reshape((1,))
    return _sc_copy(nst, pos_emb_weight)
